# baseline (speedup 1.0000x reference)
import functools

import jax
import jax.numpy as jnp
from jax import lax
from jax.experimental import pallas as pl
from jax.experimental.pallas import tpu as pltpu, tpu_sc as plsc

B = 4
S = 2048
D = 1024
N_ROWS = B * S
NC, NS = 2, 16
NW = NC * NS
SC_ROWS = 2048
ROWS_PER_W = SC_ROWS // NW
CHUNK = 16
NCHUNK = ROWS_PER_W // CHUNK
NSLOT = 3 if NCHUNK >= 3 else 2
RB = 512
TC_BLK0 = SC_ROWS // RB
TC_GRID = (N_ROWS - SC_ROWS) // RB


def _sc_body(x_hbm, pe_hbm, i0_hbm, i1_hbm, i2_hbm, out_hbm,
             i0_v, i1_v, i2_v, *bufs_and_sems):
    x_bufs = bufs_and_sems[0:NSLOT]
    pe_bufs = bufs_and_sems[NSLOT:2 * NSLOT]
    sem_x = bufs_and_sems[2 * NSLOT:3 * NSLOT]
    sem_pe = bufs_and_sems[3 * NSLOT:4 * NSLOT]
    sem_o = bufs_and_sems[4 * NSLOT:5 * NSLOT]

    wid = lax.axis_index("s") * NC + lax.axis_index("c")
    row_base = wid * ROWS_PER_W
    b = lax.shift_right_logical(row_base, 11)
    s_base = row_base & (S - 1)

    pltpu.sync_copy(i0_hbm, i0_v)
    pltpu.sync_copy(i1_hbm, i1_v)
    pltpu.sync_copy(i2_hbm, i2_v)

    lane = lax.iota(jnp.int32, 16)
    b16 = lax.broadcast(b * 16, (16,))
    b8 = lax.broadcast(b * 8, (16,))

    def make_t(c):
        s = s_base + c * CHUNK + lane
        i = lax.shift_right_logical(s, 7)
        j = lax.shift_right_logical(s, 3) & 15
        k = s & 7
        return (plsc.load_gather(i0_v, [b16 + i])
                + plsc.load_gather(i1_v, [b16 + j])
                + plsc.load_gather(i2_v, [b8 + k]))

    handles = {}

    def start_chunk(c):
        slot = c % NSLOT
        if c >= NSLOT:
            handles[("o", c - NSLOT)].wait()
        t = make_t(c)
        handles[("pe", c)] = pltpu.async_copy(
            pe_hbm.at[t], pe_bufs[slot], sem_pe[slot])
        handles[("x", c)] = pltpu.async_copy(
            x_hbm.at[pl.ds(row_base + c * CHUNK, CHUNK)], x_bufs[slot],
            sem_x[slot])

    for c in range(min(NSLOT - 1, NCHUNK)):
        start_chunk(c)
    for c in range(NCHUNK):
        slot = c % NSLOT
        if c + NSLOT - 1 < NCHUNK:
            start_chunk(c + NSLOT - 1)
        handles[("x", c)].wait()
        handles[("pe", c)].wait()
        xb = x_bufs[slot]
        pb = pe_bufs[slot]

        def add_body(g, _, xb=xb, pb=pb):
            col = g * 16
            for r in range(CHUNK):
                xb[r, pl.ds(col, 16)] = xb[r, pl.ds(col, 16)] + pb[r, pl.ds(col, 16)]
            return _

        lax.fori_loop(0, D // 16, add_body, None)
        handles[("o", c)] = pltpu.async_copy(
            xb, out_hbm.at[pl.ds(row_base + c * CHUNK, CHUNK)], sem_o[slot])
    for c in range(max(0, NCHUNK - NSLOT), NCHUNK):
        handles[("o", c)].wait()


@jax.jit
def _sc_call(x2d, pe, idxs_0, idxs_1, idxs_2):
    mesh = plsc.VectorSubcoreMesh(
        core_axis_name="c", subcore_axis_name="s",
        num_cores=NC, num_subcores=NS)
    fn = pl.kernel(
        _sc_body,
        out_type=jax.ShapeDtypeStruct((N_ROWS, D), jnp.float32),
        mesh=mesh,
        compiler_params=pltpu.CompilerParams(needs_layout_passes=False),
        scratch_types=(
            [pltpu.VMEM((B * 16,), jnp.int32),
             pltpu.VMEM((B * 16,), jnp.int32),
             pltpu.VMEM((B * 8,), jnp.int32)]
            + [pltpu.VMEM((CHUNK, D), jnp.float32)] * (2 * NSLOT)
            + [pltpu.SemaphoreType.DMA] * (3 * NSLOT)
        ),
    )
    return fn(x2d, pe, idxs_0, idxs_1, idxs_2)


def _tc_body(i0_s, i1_s, i2_s, x_ref, pe_ref, sc_ref, out_ref):
    del sc_ref
    blk = pl.program_id(0)
    gr0 = SC_ROWS + blk * RB

    def row(r, carry):
        gr = gr0 + r
        b = lax.shift_right_logical(gr, 11)
        s = gr & (S - 1)
        t = (i0_s[b, lax.shift_right_logical(s, 7)]
             + i1_s[b, lax.shift_right_logical(s, 3) & 15]
             + i2_s[b, s & 7])
        out_ref[r] = x_ref[r] + pe_ref[t]
        return carry

    lax.fori_loop(0, RB, row, None, unroll=8)


@jax.jit
def _tc_call(x3, pe3, idxs_0, idxs_1, idxs_2, sc_out3):
    return pl.pallas_call(
        _tc_body,
        grid=(TC_GRID,),
        in_specs=[
            pl.BlockSpec(memory_space=pltpu.SMEM),
            pl.BlockSpec(memory_space=pltpu.SMEM),
            pl.BlockSpec(memory_space=pltpu.SMEM),
            pl.BlockSpec((RB, 8, 128), lambda i: (TC_BLK0 + i, 0, 0)),
            pl.BlockSpec((2000, 8, 128), lambda i: (0, 0, 0)),
            pl.BlockSpec(memory_space=pl.ANY),
        ],
        out_specs=pl.BlockSpec((RB, 8, 128), lambda i: (TC_BLK0 + i, 0, 0)),
        out_shape=jax.ShapeDtypeStruct((N_ROWS, 8, 128), jnp.float32),
        input_output_aliases={5: 0},
        compiler_params=pltpu.CompilerParams(
            dimension_semantics=("arbitrary",)),
    )(idxs_0, idxs_1, idxs_2, x3, pe3, sc_out3)


def kernel(x, pe, idxs_0, idxs_1, idxs_2):
    x2d = x.reshape(N_ROWS, D)
    sc_out = _sc_call(x2d, pe, idxs_0.reshape(-1), idxs_1.reshape(-1),
                      idxs_2.reshape(-1))
    out3 = _tc_call(x2d.reshape(N_ROWS, 8, 128), pe.reshape(2000, 8, 128),
                    idxs_0, idxs_1, idxs_2, sc_out.reshape(N_ROWS, 8, 128))
    return out3.reshape(B, S, D)


# speedup vs baseline: 1.1987x; 1.1987x over previous
import functools

import jax
import jax.numpy as jnp
from jax import lax
from jax.experimental import pallas as pl
from jax.experimental.pallas import tpu as pltpu, tpu_sc as plsc

B = 4
S = 2048
D = 1024
N_ROWS = B * S
NC, NS = 2, 16
NW = NC * NS
SC_ROWS = 2048
ROWS_PER_W = SC_ROWS // NW
CHUNK = 16
NCHUNK = ROWS_PER_W // CHUNK
NSLOT = 3 if NCHUNK >= 3 else 2
RB = 512
TC_BLK0 = SC_ROWS // RB
TC_GRID = (N_ROWS - SC_ROWS) // RB


def _sc_body(x_hbm, pe_hbm, i0_hbm, i1_hbm, i2_hbm, out_hbm,
             i0_v, i1_v, i2_v, *bufs_and_sems):
    x_bufs = bufs_and_sems[0:NSLOT]
    pe_bufs = bufs_and_sems[NSLOT:2 * NSLOT]
    sem_x = bufs_and_sems[2 * NSLOT:3 * NSLOT]
    sem_pe = bufs_and_sems[3 * NSLOT:4 * NSLOT]
    sem_o = bufs_and_sems[4 * NSLOT:5 * NSLOT]

    wid = lax.axis_index("s") * NC + lax.axis_index("c")
    row_base = wid * ROWS_PER_W
    b = lax.shift_right_logical(row_base, 11)
    s_base = row_base & (S - 1)

    pltpu.sync_copy(i0_hbm, i0_v)
    pltpu.sync_copy(i1_hbm, i1_v)
    pltpu.sync_copy(i2_hbm, i2_v)

    lane = lax.iota(jnp.int32, 16)
    b16 = lax.broadcast(b * 16, (16,))
    b8 = lax.broadcast(b * 8, (16,))

    def make_t(c):
        s = s_base + c * CHUNK + lane
        i = lax.shift_right_logical(s, 7)
        j = lax.shift_right_logical(s, 3) & 15
        k = s & 7
        return (plsc.load_gather(i0_v, [b16 + i])
                + plsc.load_gather(i1_v, [b16 + j])
                + plsc.load_gather(i2_v, [b8 + k]))

    handles = {}

    def start_chunk(c):
        slot = c % NSLOT
        if c >= NSLOT:
            handles[("o", c - NSLOT)].wait()
        t = make_t(c)
        handles[("pe", c)] = pltpu.async_copy(
            pe_hbm.at[t], pe_bufs[slot], sem_pe[slot])
        handles[("x", c)] = pltpu.async_copy(
            x_hbm.at[pl.ds(row_base + c * CHUNK, CHUNK)], x_bufs[slot],
            sem_x[slot])

    for c in range(min(NSLOT - 1, NCHUNK)):
        start_chunk(c)
    for c in range(NCHUNK):
        slot = c % NSLOT
        if c + NSLOT - 1 < NCHUNK:
            start_chunk(c + NSLOT - 1)
        handles[("x", c)].wait()
        handles[("pe", c)].wait()
        xb = x_bufs[slot]
        pb = pe_bufs[slot]

        def add_body(g, _, xb=xb, pb=pb):
            col = g * 16
            for r in range(CHUNK):
                xb[r, pl.ds(col, 16)] = xb[r, pl.ds(col, 16)] + pb[r, pl.ds(col, 16)]
            return _

        lax.fori_loop(0, D // 16, add_body, None)
        handles[("o", c)] = pltpu.async_copy(
            xb, out_hbm.at[pl.ds(row_base + c * CHUNK, CHUNK)], sem_o[slot])
    for c in range(max(0, NCHUNK - NSLOT), NCHUNK):
        handles[("o", c)].wait()


@jax.jit
def _sc_call(x2d, pe, idxs_0, idxs_1, idxs_2):
    mesh = plsc.VectorSubcoreMesh(
        core_axis_name="c", subcore_axis_name="s",
        num_cores=NC, num_subcores=NS)
    fn = pl.kernel(
        _sc_body,
        out_type=jax.ShapeDtypeStruct((N_ROWS, D), jnp.float32),
        mesh=mesh,
        compiler_params=pltpu.CompilerParams(needs_layout_passes=False),
        scratch_types=(
            [pltpu.VMEM((B * 16,), jnp.int32),
             pltpu.VMEM((B * 16,), jnp.int32),
             pltpu.VMEM((B * 8,), jnp.int32)]
            + [pltpu.VMEM((CHUNK, D), jnp.float32)] * (2 * NSLOT)
            + [pltpu.SemaphoreType.DMA] * (3 * NSLOT)
        ),
    )
    return fn(x2d, pe, idxs_0, idxs_1, idxs_2)


def _tc_body(i0_s, i1_s, i2_s, x_ref, pe_ref, sc_ref, out_ref):
    del sc_ref
    blk = pl.program_id(0)
    gr0 = SC_ROWS + blk * RB
    b = lax.shift_right_logical(gr0, 11)
    s0 = gr0 & (S - 1)

    def group(jj, carry):
        s_grp = s0 + jj * 8
        base = (i0_s[b, lax.shift_right_logical(s_grp, 7)]
                + i1_s[b, lax.shift_right_logical(s_grp, 3) & 15])
        r0 = jj * 8
        for k in range(8):
            t = base + i2_s[b, k]
            out_ref[r0 + k] = x_ref[r0 + k] + pe_ref[t]
        return carry

    lax.fori_loop(0, RB // 8, group, None, unroll=4)


@jax.jit
def _tc_call(x3, pe3, idxs_0, idxs_1, idxs_2, sc_out3):
    return pl.pallas_call(
        _tc_body,
        grid=(TC_GRID,),
        in_specs=[
            pl.BlockSpec(memory_space=pltpu.SMEM),
            pl.BlockSpec(memory_space=pltpu.SMEM),
            pl.BlockSpec(memory_space=pltpu.SMEM),
            pl.BlockSpec((RB, 8, 128), lambda i: (TC_BLK0 + i, 0, 0)),
            pl.BlockSpec((2000, 8, 128), lambda i: (0, 0, 0)),
            pl.BlockSpec(memory_space=pl.ANY),
        ],
        out_specs=pl.BlockSpec((RB, 8, 128), lambda i: (TC_BLK0 + i, 0, 0)),
        out_shape=jax.ShapeDtypeStruct((N_ROWS, 8, 128), jnp.float32),
        input_output_aliases={5: 0},
        compiler_params=pltpu.CompilerParams(
            dimension_semantics=("arbitrary",)),
    )(idxs_0, idxs_1, idxs_2, x3, pe3, sc_out3)


def kernel(x, pe, idxs_0, idxs_1, idxs_2):
    x2d = x.reshape(N_ROWS, D)
    sc_out = _sc_call(x2d, pe, idxs_0.reshape(-1), idxs_1.reshape(-1),
                      idxs_2.reshape(-1))
    out3 = _tc_call(x2d.reshape(N_ROWS, 8, 128), pe.reshape(2000, 8, 128),
                    idxs_0, idxs_1, idxs_2, sc_out.reshape(N_ROWS, 8, 128))
    return out3.reshape(B, S, D)


# speedup vs baseline: 2.3526x; 1.9625x over previous
import functools

import jax
import jax.numpy as jnp
from jax import lax
from jax.experimental import pallas as pl
from jax.experimental.pallas import tpu as pltpu, tpu_sc as plsc

B = 4
S = 2048
D = 1024
NC, NS = 2, 16
NW = NC * NS
ROWS_PER_W = (B * S) // NW
CHUNK = 16
NCHUNK = ROWS_PER_W // CHUNK
WORKERS_PER_B = NW // B
NSLOT = 3


def _sc_body(x_hbm, pe_hbm, i0_hbm, i1_hbm, i2_hbm, out_hbm,
             i0_v, i1_v, i2_v, *bufs_and_sems):
    x_bufs = bufs_and_sems[0:NSLOT]
    pe_bufs = bufs_and_sems[NSLOT:2 * NSLOT]
    sem_x = bufs_and_sems[2 * NSLOT:3 * NSLOT]
    sem_pe = bufs_and_sems[3 * NSLOT:4 * NSLOT]
    sem_o = bufs_and_sems[4 * NSLOT:5 * NSLOT]

    wid = lax.axis_index("s") * NC + lax.axis_index("c")
    b = wid // WORKERS_PER_B
    s_base = (wid % WORKERS_PER_B) * ROWS_PER_W
    row_base = wid * ROWS_PER_W

    pltpu.sync_copy(i0_hbm, i0_v)
    pltpu.sync_copy(i1_hbm, i1_v)
    pltpu.sync_copy(i2_hbm, i2_v)

    lane = lax.iota(jnp.int32, 16)
    b16 = lax.broadcast(b * 16, (16,))
    b8 = lax.broadcast(b * 8, (16,))

    def make_t(c):
        s = s_base + c * CHUNK + lane
        i = lax.shift_right_logical(s, 7)
        j = lax.shift_right_logical(s, 3) & 15
        k = s & 7
        return (plsc.load_gather(i0_v, [b16 + i])
                + plsc.load_gather(i1_v, [b16 + j])
                + plsc.load_gather(i2_v, [b8 + k]))

    handles = {}

    def start_chunk(c):
        slot = c % NSLOT
        if c >= NSLOT:
            handles[("o", c - NSLOT, 0)].wait()
            handles[("o", c - NSLOT, 1)].wait()
        t = make_t(c)
        handles[("pe", c)] = pltpu.async_copy(
            pe_hbm.at[t], pe_bufs[slot], sem_pe[slot])
        handles[("x", c)] = pltpu.async_copy(
            x_hbm.at[pl.ds(row_base + c * CHUNK, CHUNK)], x_bufs[slot],
            sem_x[slot])

    start_chunk(0)
    start_chunk(1)
    for c in range(NCHUNK):
        slot = c % NSLOT
        if c + 2 < NCHUNK:
            start_chunk(c + 2)
        handles[("x", c)].wait()
        handles[("pe", c)].wait()
        xb = x_bufs[slot]
        pb = pe_bufs[slot]

        half = CHUNK // 2
        for h in range(2):
            r0 = h * half

            def add_body(g, _, xb=xb, pb=pb, r0=r0):
                col = g * 16
                for r in range(r0, r0 + half):
                    xb[r, pl.ds(col, 16)] = (
                        xb[r, pl.ds(col, 16)] + pb[r, pl.ds(col, 16)])
                return _

            lax.fori_loop(0, D // 16, add_body, None, unroll=2)
            handles[("o", c, h)] = pltpu.async_copy(
                xb.at[pl.ds(r0, half)],
                out_hbm.at[pl.ds(row_base + c * CHUNK + r0, half)],
                sem_o[slot])
    for c in range(NCHUNK - NSLOT, NCHUNK):
        handles[("o", c, 0)].wait()
        handles[("o", c, 1)].wait()


@jax.jit
def _sc_call(x2d, pe, idxs_0, idxs_1, idxs_2):
    mesh = plsc.VectorSubcoreMesh(
        core_axis_name="c", subcore_axis_name="s",
        num_cores=NC, num_subcores=NS)
    fn = pl.kernel(
        _sc_body,
        out_type=jax.ShapeDtypeStruct((B * S, D), jnp.float32),
        mesh=mesh,
        compiler_params=pltpu.CompilerParams(needs_layout_passes=False),
        scratch_types=(
            [pltpu.VMEM((B * 16,), jnp.int32),
             pltpu.VMEM((B * 16,), jnp.int32),
             pltpu.VMEM((B * 8,), jnp.int32)]
            + [pltpu.VMEM((CHUNK, D), jnp.float32)] * (2 * NSLOT)
            + [pltpu.SemaphoreType.DMA] * (3 * NSLOT)
        ),
    )
    return fn(x2d, pe, idxs_0, idxs_1, idxs_2)


def kernel(x, pe, idxs_0, idxs_1, idxs_2):
    out = _sc_call(x.reshape(B * S, D), pe, idxs_0.reshape(-1),
                   idxs_1.reshape(-1), idxs_2.reshape(-1))
    return out.reshape(B, S, D)
